# baseline (speedup 1.0000x reference)
import functools

import jax
import jax.numpy as jnp
import numpy as np
from jax import lax
from jax.experimental import pallas as pl
from jax.experimental.pallas import tpu as pltpu
from jax.experimental.pallas import tpu_sc as plsc

B = 1024
F = 256
BOARD = 15
HW = BOARD * BOARD
PCODE_DIM = 2380
EMBED_DIM = 2 * (PCODE_DIM + 1)
NCELL = B * HW
NC = 2
NS = 16
NW = NC * NS
CPT = NCELL // NW
CH = 80
NCHUNK = CPT // CH
LANES = 16
FV = F // LANES


def _offset_map_flat():
    bd = BOARD // 3
    m = np.zeros((BOARD, BOARD), dtype=np.int32)
    for y in range(BOARD):
        for x in range(BOARD):
            m[y, x] = (y // 3) * bd + (x // 3)
    return jnp.asarray((m * EMBED_DIM).reshape(-1))


def _sc_gather_sum(pcode, symboard, idx4):
    mesh = plsc.VectorSubcoreMesh(core_axis_name="c", subcore_axis_name="s")

    @functools.partial(
        pl.kernel,
        out_type=jax.ShapeDtypeStruct((NCELL, F), jnp.float32),
        mesh=mesh,
        scratch_types=[
            pltpu.VMEM((4 * NCHUNK, CH), jnp.int32),
            pltpu.VMEM((CH, F), jnp.float32),
            pltpu.VMEM((CH, F), jnp.float32),
            pltpu.VMEM((CH, F), jnp.float32),
            pltpu.VMEM((CH, F), jnp.float32),
            pltpu.SemaphoreType.DMA,
        ],
    )
    def k(pcode_hbm, sym_hbm, idx_hbm, out_hbm, idx_v, ba, bb, bc, bd, sem):
        wid = lax.axis_index("s") * NC + lax.axis_index("c")
        base = wid * CPT
        pltpu.sync_copy(idx_hbm.at[wid], idx_v)

        def body(j, carry):
            ca = pltpu.async_copy(pcode_hbm.at[idx_v.at[j]], ba, sem)
            cb = pltpu.async_copy(pcode_hbm.at[idx_v.at[NCHUNK + j]], bb, sem)
            cc = pltpu.async_copy(sym_hbm.at[idx_v.at[2 * NCHUNK + j]], bc, sem)
            cd = pltpu.async_copy(sym_hbm.at[idx_v.at[3 * NCHUNK + j]], bd, sem)
            ca.wait()
            cb.wait()
            cc.wait()
            cd.wait()

            def add_row(r, carry2):
                for kk in range(FV):
                    sl = pl.ds(kk * LANES, LANES)
                    ba[r, sl] = (ba[r, sl] + bb[r, sl]) + (bc[r, sl] + bd[r, sl])
                return carry2

            lax.fori_loop(0, CH, add_row, 0)
            pltpu.sync_copy(ba, out_hbm.at[pl.ds(base + j * CH, CH)])
            return carry

        lax.fori_loop(0, NCHUNK, body, 0)

    return k(pcode, symboard, idx4)


def kernel(pcode_table, symboard_table, sparse_feature_dim,
           sparse_feature_input, board_input):
    off = _offset_map_flat()[None, :]
    p0 = jnp.where(board_input[:, 0] > 0, PCODE_DIM,
                   sparse_feature_input[:, 10]).reshape(B, HW)
    p1 = jnp.where(board_input[:, 1] > 0, PCODE_DIM,
                   sparse_feature_input[:, 11]).reshape(B, HW) + (PCODE_DIM + 1)
    idx4 = jnp.stack([p0, p1, p0 + off, p1 + off]).astype(jnp.int32)
    idx4 = idx4.reshape(4, NW, NCHUNK, CH).transpose(1, 0, 2, 3)
    idx4 = idx4.reshape(NW, 4 * NCHUNK, CH)
    inter = _sc_gather_sum(pcode_table, symboard_table, idx4)
    return inter.reshape(B, BOARD, BOARD, F).transpose(0, 3, 1, 2)


# speedup vs baseline: 3.8679x; 3.8679x over previous
import functools

import jax
import jax.numpy as jnp
import numpy as np
from jax import lax
from jax.experimental import pallas as pl
from jax.experimental.pallas import tpu as pltpu
from jax.experimental.pallas import tpu_sc as plsc

B = 1024
F = 256
BOARD = 15
HW = BOARD * BOARD
PCODE_DIM = 2380
EMBED_DIM = 2 * (PCODE_DIM + 1)
NBLOCK = (BOARD // 3) ** 2
NCELL = B * HW
NC = 2
NS = 16
NW = NC * NS
CPT = NCELL // NW
CH = 48
NCHUNK = CPT // CH
LANES = 16
FV = F // LANES


def _offset_map_flat():
    bd = BOARD // 3
    m = np.zeros((BOARD, BOARD), dtype=np.int32)
    for y in range(BOARD):
        for x in range(BOARD):
            m[y, x] = (y // 3) * bd + (x // 3)
    return jnp.asarray((m * EMBED_DIM).reshape(-1))


def _build_fused(pcode, symboard):
    sym3 = symboard.reshape(NBLOCK, EMBED_DIM, F)

    def body(sym_ref, pc_ref, out_ref):
        out_ref[...] = sym_ref[...] + pc_ref[...][None]

    fused3 = pl.pallas_call(
        body,
        grid=(NBLOCK,),
        in_specs=[
            pl.BlockSpec((1, EMBED_DIM, F), lambda k: (k, 0, 0)),
            pl.BlockSpec((EMBED_DIM, F), lambda k: (0, 0)),
        ],
        out_specs=pl.BlockSpec((1, EMBED_DIM, F), lambda k: (k, 0, 0)),
        out_shape=jax.ShapeDtypeStruct((NBLOCK, EMBED_DIM, F), jnp.float32),
    )(sym3, pcode)
    return fused3.reshape(NBLOCK * EMBED_DIM, F)


def _sc_gather_sum(fused, idx2):
    mesh = plsc.VectorSubcoreMesh(core_axis_name="c", subcore_axis_name="s")

    @functools.partial(
        pl.kernel,
        out_type=jax.ShapeDtypeStruct((NCELL, F), jnp.float32),
        mesh=mesh,
        scratch_types=[
            pltpu.VMEM((NCHUNK, 2 * CH), jnp.int32),
            pltpu.VMEM((2 * CH, F), jnp.float32),
            pltpu.VMEM((2 * CH, F), jnp.float32),
            pltpu.VMEM((CH, F), jnp.float32),
            pltpu.VMEM((CH, F), jnp.float32),
            pltpu.SemaphoreType.DMA,
            pltpu.SemaphoreType.DMA,
            pltpu.SemaphoreType.DMA,
            pltpu.SemaphoreType.DMA,
        ],
    )
    def k(fused_hbm, idx_hbm, out_hbm, idx_v, b0, b1, o0, o1,
          sg0, sg1, sw0, sw1):
        wid = lax.axis_index("s") * NC + lax.axis_index("c")
        base = wid * CPT
        pltpu.sync_copy(idx_hbm.at[wid], idx_v)

        pltpu.async_copy(fused_hbm.at[idx_v.at[0]], b0, sg0)
        pltpu.async_copy(fused_hbm.at[idx_v.at[1]], b1, sg1)

        def halfstep(j, b, o, sg, sw):
            pltpu.make_async_copy(fused_hbm.at[idx_v.at[0]], b, sg).wait()

            @pl.when(j >= 2)
            def _():
                pltpu.make_async_copy(o, out_hbm.at[pl.ds(0, CH)], sw).wait()

            def add_row(r, carry):
                for kk in range(FV):
                    sl = pl.ds(kk * LANES, LANES)
                    o[r, sl] = b[r, sl] + b[CH + r, sl]
                return carry

            lax.fori_loop(0, CH, add_row, 0)

            @pl.when(j + 2 < NCHUNK)
            def _():
                pltpu.async_copy(fused_hbm.at[idx_v.at[j + 2]], b, sg)

            pltpu.async_copy(o, out_hbm.at[pl.ds(base + j * CH, CH)], sw)

        def body(j2, carry):
            halfstep(2 * j2, b0, o0, sg0, sw0)
            halfstep(2 * j2 + 1, b1, o1, sg1, sw1)
            return carry

        lax.fori_loop(0, NCHUNK // 2, body, 0)
        pltpu.make_async_copy(o0, out_hbm.at[pl.ds(0, CH)], sw0).wait()
        pltpu.make_async_copy(o1, out_hbm.at[pl.ds(0, CH)], sw1).wait()

    return k(fused, idx2)


def kernel(pcode_table, symboard_table, sparse_feature_dim,
           sparse_feature_input, board_input):
    off = _offset_map_flat()[None, :]
    p0 = jnp.where(board_input[:, 0] > 0, PCODE_DIM,
                   sparse_feature_input[:, 10]).reshape(B, HW)
    p1 = jnp.where(board_input[:, 1] > 0, PCODE_DIM,
                   sparse_feature_input[:, 11]).reshape(B, HW) + (PCODE_DIM + 1)
    i2 = (p0 + off).astype(jnp.int32).reshape(NCELL)
    i3 = (p1 + off).astype(jnp.int32).reshape(NCELL)
    idx2 = jnp.stack(
        [i2.reshape(NW, NCHUNK, CH), i3.reshape(NW, NCHUNK, CH)], axis=2)
    idx2 = idx2.reshape(NW, NCHUNK, 2 * CH)
    fused = _build_fused(pcode_table, symboard_table)
    inter = _sc_gather_sum(fused, idx2)
    return inter.reshape(B, BOARD, BOARD, F).transpose(0, 3, 1, 2)


# speedup vs baseline: 3.9162x; 1.0125x over previous
import functools

import jax
import jax.numpy as jnp
import numpy as np
from jax import lax
from jax.experimental import pallas as pl
from jax.experimental.pallas import tpu as pltpu
from jax.experimental.pallas import tpu_sc as plsc

B = 1024
F = 256
BOARD = 15
HW = BOARD * BOARD
PCODE_DIM = 2380
EMBED_DIM = 2 * (PCODE_DIM + 1)
EMBED_PAD = EMBED_DIM + 6
NBLOCK = (BOARD // 3) ** 2
NCELL = B * HW
NC = 2
NS = 16
NW = NC * NS
CPT = NCELL // NW
CH = 48
NCHUNK = CPT // CH
NBUF = 3
LANES = 16
FV = F // LANES


def _offset_map_flat():
    bd = BOARD // 3
    m = np.zeros((BOARD, BOARD), dtype=np.int32)
    for y in range(BOARD):
        for x in range(BOARD):
            m[y, x] = (y // 3) * bd + (x // 3)
    return jnp.asarray((m * EMBED_PAD).reshape(-1))


def _build_fused(pcode, symboard):
    sym3 = symboard.reshape(NBLOCK, EMBED_DIM, F)

    def body(sym_ref, pc_ref, out_ref):
        out_ref[0, :EMBED_DIM] = sym_ref[0] + pc_ref[...]
        out_ref[0, EMBED_DIM:] = jnp.zeros(
            (EMBED_PAD - EMBED_DIM, F), jnp.float32)

    fused3 = pl.pallas_call(
        body,
        grid=(NBLOCK,),
        in_specs=[
            pl.BlockSpec((1, EMBED_DIM, F), lambda k: (k, 0, 0)),
            pl.BlockSpec((EMBED_DIM, F), lambda k: (0, 0)),
        ],
        out_specs=pl.BlockSpec((1, EMBED_PAD, F), lambda k: (k, 0, 0)),
        out_shape=jax.ShapeDtypeStruct((NBLOCK, EMBED_PAD, F), jnp.float32),
    )(sym3, pcode)
    return fused3.reshape(NBLOCK * EMBED_PAD, F)


def _sc_gather_sum(fused, idx2):
    mesh = plsc.VectorSubcoreMesh(core_axis_name="c", subcore_axis_name="s")

    @functools.partial(
        pl.kernel,
        out_type=jax.ShapeDtypeStruct((NCELL, F), jnp.float32),
        mesh=mesh,
        scratch_types=[
            pltpu.VMEM((NCHUNK, 2 * CH), jnp.int32),
            [pltpu.VMEM((2 * CH, F), jnp.float32)] * NBUF,
            [pltpu.VMEM((CH, F), jnp.float32)] * NBUF,
            [pltpu.SemaphoreType.DMA] * NBUF,
            [pltpu.SemaphoreType.DMA] * NBUF,
        ],
    )
    def k(fused_hbm, idx_hbm, out_hbm, idx_v, bufs, outs, sgs, sws):
        wid = lax.axis_index("s") * NC + lax.axis_index("c")
        base = wid * CPT
        pltpu.sync_copy(idx_hbm.at[wid], idx_v)

        for p in range(NBUF):
            pltpu.async_copy(fused_hbm.at[idx_v.at[p]], bufs[p], sgs[p])

        def halfstep(j, b, o, sg, sw):
            pltpu.make_async_copy(fused_hbm.at[idx_v.at[0]], b, sg).wait()

            @pl.when(j >= NBUF)
            def _():
                pltpu.make_async_copy(o, out_hbm.at[pl.ds(0, CH)], sw).wait()

            def add_row(r, carry):
                for kk in range(FV):
                    sl = pl.ds(kk * LANES, LANES)
                    o[r, sl] = b[r, sl] + b[CH + r, sl]
                return carry

            lax.fori_loop(0, CH, add_row, 0)

            @pl.when(j + NBUF < NCHUNK)
            def _():
                pltpu.async_copy(fused_hbm.at[idx_v.at[j + NBUF]], b, sg)

            pltpu.async_copy(o, out_hbm.at[pl.ds(base + j * CH, CH)], sw)

        def body(jj, carry):
            for p in range(NBUF):
                halfstep(NBUF * jj + p, bufs[p], outs[p], sgs[p], sws[p])
            return carry

        lax.fori_loop(0, NCHUNK // NBUF, body, 0)
        for p in range(NBUF):
            pltpu.make_async_copy(outs[p], out_hbm.at[pl.ds(0, CH)],
                                  sws[p]).wait()

    return k(fused, idx2)


def kernel(pcode_table, symboard_table, sparse_feature_dim,
           sparse_feature_input, board_input):
    off = _offset_map_flat()[None, :]
    p0 = jnp.where(board_input[:, 0] > 0, PCODE_DIM,
                   sparse_feature_input[:, 10]).reshape(B, HW)
    p1 = jnp.where(board_input[:, 1] > 0, PCODE_DIM,
                   sparse_feature_input[:, 11]).reshape(B, HW) + (PCODE_DIM + 1)
    i2 = (p0 + off).astype(jnp.int32).reshape(NCELL)
    i3 = (p1 + off).astype(jnp.int32).reshape(NCELL)
    idx2 = jnp.stack(
        [i2.reshape(NW, NCHUNK, CH), i3.reshape(NW, NCHUNK, CH)], axis=2)
    idx2 = idx2.reshape(NW, NCHUNK, 2 * CH)
    fused = _build_fused(pcode_table, symboard_table)
    inter = _sc_gather_sum(fused, idx2)
    return inter.reshape(B, BOARD, BOARD, F).transpose(0, 3, 1, 2)


# speedup vs baseline: 4.3498x; 1.1107x over previous
import functools

import jax
import jax.numpy as jnp
import numpy as np
from jax import lax
from jax.experimental import pallas as pl
from jax.experimental.pallas import tpu as pltpu
from jax.experimental.pallas import tpu_sc as plsc

B = 1024
F = 256
BOARD = 15
HW = BOARD * BOARD
PCODE_DIM = 2380
EMBED_DIM = 2 * (PCODE_DIM + 1)
EMBED_PAD = EMBED_DIM + 6
NBLOCK = (BOARD // 3) ** 2
NCELL = B * HW
NC = 2
NS = 16
NW = NC * NS
CPT = NCELL // NW
CH = 48
NCHUNK = CPT // CH
NBUF = 2
LANES = 16
FV = F // LANES


def _offset_map_flat():
    bd = BOARD // 3
    m = np.zeros((BOARD, BOARD), dtype=np.int32)
    for y in range(BOARD):
        for x in range(BOARD):
            m[y, x] = (y // 3) * bd + (x // 3)
    return jnp.asarray((m * EMBED_PAD).reshape(-1))


def _build_fused(pcode, symboard):
    sym3 = symboard.reshape(NBLOCK, EMBED_DIM, F)

    def body(sym_ref, pc_ref, out_ref):
        out_ref[0, :EMBED_DIM] = sym_ref[0] + pc_ref[...]
        out_ref[0, EMBED_DIM:] = jnp.zeros(
            (EMBED_PAD - EMBED_DIM, F), jnp.float32)

    fused3 = pl.pallas_call(
        body,
        grid=(NBLOCK,),
        in_specs=[
            pl.BlockSpec((1, EMBED_DIM, F), lambda k: (k, 0, 0)),
            pl.BlockSpec((EMBED_DIM, F), lambda k: (0, 0)),
        ],
        out_specs=pl.BlockSpec((1, EMBED_PAD, F), lambda k: (k, 0, 0)),
        out_shape=jax.ShapeDtypeStruct((NBLOCK, EMBED_PAD, F), jnp.float32),
    )(sym3, pcode)
    return fused3.reshape(NBLOCK * EMBED_PAD, F)


def _sc_gather_sum(fused, idx2):
    mesh = plsc.VectorSubcoreMesh(core_axis_name="c", subcore_axis_name="s")

    @functools.partial(
        pl.kernel,
        out_type=jax.ShapeDtypeStruct((NCELL, F), jnp.float32),
        mesh=mesh,
        scratch_types=[
            pltpu.VMEM((NCHUNK, 2 * CH), jnp.int32),
            [pltpu.VMEM((2 * CH, F), jnp.float32)] * NBUF,
            [pltpu.VMEM((CH, F), jnp.float32)] * NBUF,
            [pltpu.SemaphoreType.DMA] * NBUF,
            [pltpu.SemaphoreType.DMA] * NBUF,
        ],
    )
    def k(fused_hbm, idx_hbm, out_hbm, idx_v, bufs, outs, sgs, sws):
        wid = lax.axis_index("s") * NC + lax.axis_index("c")
        base = wid * CPT
        pltpu.sync_copy(idx_hbm.at[wid], idx_v)

        for p in range(NBUF):
            pltpu.async_copy(fused_hbm.at[idx_v.at[p]], bufs[p], sgs[p])

        def halfstep(j, b, o, sg, sw):
            pltpu.make_async_copy(fused_hbm.at[idx_v.at[0]], b, sg).wait()

            @pl.when(j >= NBUF)
            def _():
                pltpu.make_async_copy(o, out_hbm.at[pl.ds(0, CH)], sw).wait()

            def add_row(r, carry):
                for kk in range(FV):
                    sl = pl.ds(kk * LANES, LANES)
                    o[r, sl] = b[r, sl] + b[CH + r, sl]
                return carry

            lax.fori_loop(0, CH, add_row, 0)

            @pl.when(j + NBUF < NCHUNK)
            def _():
                pltpu.async_copy(fused_hbm.at[idx_v.at[j + NBUF]], b, sg)

            pltpu.async_copy(o, out_hbm.at[pl.ds(base + j * CH, CH)], sw)

        def body(jj, carry):
            for p in range(NBUF):
                halfstep(NBUF * jj + p, bufs[p], outs[p], sgs[p], sws[p])
            return carry

        lax.fori_loop(0, NCHUNK // NBUF, body, 0)
        for p in range(NBUF):
            pltpu.make_async_copy(outs[p], out_hbm.at[pl.ds(0, CH)],
                                  sws[p]).wait()

    return k(fused, idx2)


def kernel(pcode_table, symboard_table, sparse_feature_dim,
           sparse_feature_input, board_input):
    off = _offset_map_flat()[None, :]
    p0 = jnp.where(board_input[:, 0] > 0, PCODE_DIM,
                   sparse_feature_input[:, 10]).reshape(B, HW)
    p1 = jnp.where(board_input[:, 1] > 0, PCODE_DIM,
                   sparse_feature_input[:, 11]).reshape(B, HW) + (PCODE_DIM + 1)
    i2 = (p0 + off).astype(jnp.int32).T.reshape(NCELL)
    i3 = (p1 + off).astype(jnp.int32).T.reshape(NCELL)
    idx2 = jnp.stack(
        [i2.reshape(NW, NCHUNK, CH), i3.reshape(NW, NCHUNK, CH)], axis=2)
    idx2 = idx2.reshape(NW, NCHUNK, 2 * CH)
    fused = _build_fused(pcode_table, symboard_table)
    inter = _sc_gather_sum(fused, idx2)
    return inter.reshape(BOARD, BOARD, B, F).transpose(2, 3, 0, 1)


# speedup vs baseline: 5.3880x; 1.2387x over previous
import functools

import jax
import jax.numpy as jnp
import numpy as np
from jax import lax
from jax.experimental import pallas as pl
from jax.experimental.pallas import tpu as pltpu
from jax.experimental.pallas import tpu_sc as plsc

B = 1024
F = 256
BOARD = 15
HW = BOARD * BOARD
PCODE_DIM = 2380
EMBED_DIM = 2 * (PCODE_DIM + 1)
EMBED_PAD = EMBED_DIM + 6
NBLOCK = (BOARD // 3) ** 2
NCELL = B * HW
NC = 2
NS = 16
NW = NC * NS
CPT = NCELL // NW
CH = 48
NCHUNK = CPT // CH
NBUF = 2
LANES = 16
FV = F // LANES


def _offset_map_flat():
    bd = BOARD // 3
    m = np.zeros((BOARD, BOARD), dtype=np.int32)
    for y in range(BOARD):
        for x in range(BOARD):
            m[y, x] = (y // 3) * bd + (x // 3)
    return jnp.asarray((m * EMBED_PAD).reshape(-1))


def _build_fused(pcode, symboard):
    sym3 = symboard.reshape(NBLOCK, EMBED_DIM, F)

    def body(sym_ref, pc_ref, out_ref):
        out_ref[0, :EMBED_DIM] = sym_ref[0] + pc_ref[...]
        out_ref[0, EMBED_DIM:] = jnp.zeros(
            (EMBED_PAD - EMBED_DIM, F), jnp.float32)

    fused3 = pl.pallas_call(
        body,
        grid=(NBLOCK,),
        in_specs=[
            pl.BlockSpec((1, EMBED_DIM, F), lambda k: (k, 0, 0)),
            pl.BlockSpec((EMBED_DIM, F), lambda k: (0, 0)),
        ],
        out_specs=pl.BlockSpec((1, EMBED_PAD, F), lambda k: (k, 0, 0)),
        out_shape=jax.ShapeDtypeStruct((NBLOCK, EMBED_PAD, F), jnp.float32),
    )(sym3, pcode)
    return fused3.reshape(NBLOCK * EMBED_PAD, F)


def _sc_gather_sum(fused, idx2, widx):
    mesh = plsc.VectorSubcoreMesh(core_axis_name="c", subcore_axis_name="s")

    @functools.partial(
        pl.kernel,
        out_type=jax.ShapeDtypeStruct((NCELL, F), jnp.float32),
        mesh=mesh,
        scratch_types=[
            pltpu.VMEM((NCHUNK, 2 * CH), jnp.int32),
            pltpu.VMEM((NCHUNK, CH), jnp.int32),
            [pltpu.VMEM((2 * CH, F), jnp.float32)] * NBUF,
            [pltpu.VMEM((CH, F), jnp.float32)] * NBUF,
            [pltpu.SemaphoreType.DMA] * NBUF,
            [pltpu.SemaphoreType.DMA] * NBUF,
        ],
    )
    def k(fused_hbm, idx_hbm, widx_hbm, out_hbm, idx_v, widx_v, bufs, outs,
          sgs, sws):
        wid = lax.axis_index("s") * NC + lax.axis_index("c")
        pltpu.sync_copy(idx_hbm.at[wid], idx_v)
        pltpu.sync_copy(widx_hbm.at[wid], widx_v)

        for p in range(NBUF):
            pltpu.async_copy(fused_hbm.at[idx_v.at[p]], bufs[p], sgs[p])

        def halfstep(j, b, o, sg, sw):
            pltpu.make_async_copy(fused_hbm.at[idx_v.at[0]], b, sg).wait()

            @pl.when(j >= NBUF)
            def _():
                pltpu.make_async_copy(
                    o, out_hbm.at[widx_v.at[0]], sw).wait()

            def add_row(r, carry):
                for kk in range(FV):
                    sl = pl.ds(kk * LANES, LANES)
                    o[r, sl] = b[r, sl] + b[CH + r, sl]
                return carry

            lax.fori_loop(0, CH, add_row, 0)

            @pl.when(j + NBUF < NCHUNK)
            def _():
                pltpu.async_copy(fused_hbm.at[idx_v.at[j + NBUF]], b, sg)

            pltpu.async_copy(o, out_hbm.at[widx_v.at[j]], sw)

        def body(jj, carry):
            for p in range(NBUF):
                halfstep(NBUF * jj + p, bufs[p], outs[p], sgs[p], sws[p])
            return carry

        lax.fori_loop(0, NCHUNK // NBUF, body, 0)
        for p in range(NBUF):
            pltpu.make_async_copy(outs[p], out_hbm.at[widx_v.at[0]],
                                  sws[p]).wait()

    return k(fused, idx2, widx)


def kernel(pcode_table, symboard_table, sparse_feature_dim,
           sparse_feature_input, board_input):
    off = _offset_map_flat()[None, :]
    p0 = jnp.where(board_input[:, 0] > 0, PCODE_DIM,
                   sparse_feature_input[:, 10]).reshape(B, HW)
    p1 = jnp.where(board_input[:, 1] > 0, PCODE_DIM,
                   sparse_feature_input[:, 11]).reshape(B, HW) + (PCODE_DIM + 1)
    i2 = (p0 + off).astype(jnp.int32).reshape(NCELL)
    i3 = (p1 + off).astype(jnp.int32).reshape(NCELL)
    idx2 = jnp.stack(
        [i2.reshape(NW, NCHUNK, CH), i3.reshape(NW, NCHUNK, CH)], axis=2)
    idx2 = idx2.reshape(NW, NCHUNK, 2 * CH)
    g = jnp.arange(NCELL, dtype=jnp.int32)
    widx = ((g % HW) * B + g // HW).reshape(NW, NCHUNK, CH)
    fused = _build_fused(pcode_table, symboard_table)
    inter = _sc_gather_sum(fused, idx2, widx)
    return inter.reshape(BOARD, BOARD, B, F).transpose(2, 3, 0, 1)
